# baseline (speedup 1.0000x reference)
import functools

import jax
import jax.numpy as jnp
from jax import lax
from jax.experimental import pallas as pl
from jax.experimental.pallas import tpu as pltpu, tpu_sc as plsc

IN_DIM = 1_000_000
OUT_DIM = 32
BATCH = 4096
HIST = 50

_NC = 2
_NS = 16
_NW = _NC * _NS

_B = BATCH * HIST
_U = 128
_UPW = _B // (_NW * _U)
_TCB = BATCH // 128

_NBLK = IN_DIM // 128
_BPW = -(-_NBLK // _NW)
_LAST_COLS = IN_DIM - _NBLK * 128
_GROWS = IN_DIM * OUT_DIM // 128


def _make_k1():
  mesh = plsc.VectorSubcoreMesh(core_axis_name="c", subcore_axis_name="s")

  @functools.partial(
      pl.kernel,
      out_type=jax.ShapeDtypeStruct((_GROWS, 128), jnp.float32),
      mesh=mesh,
      compiler_params=pltpu.CompilerParams(needs_layout_passes=False),
      scratch_types=[
          pltpu.VMEM((2, 32, 128), jnp.float32),
          pltpu.VMEM((32, 128), jnp.float32),
          pltpu.SemaphoreType.DMA,
          pltpu.SemaphoreType.DMA,
      ],
  )
  def k1(wt_hbm, tail_hbm, out_hbm, blk_v, chunk_v, sem0, sem1):
    wid = lax.axis_index("s") * _NC + lax.axis_index("c")
    sems = (sem0, sem1)

    def fire(i, buf):
      tc = wid + i * _NW

      @pl.when(tc < _NBLK)
      def _():
        pltpu.async_copy(
            wt_hbm.at[:, pl.ds(tc * 128, 128)], blk_v.at[buf], sems[buf])

    def drain_transpose_store(i, buf):
      tc = wid + i * _NW

      @pl.when(tc < _NBLK)
      def _():
        pltpu.make_async_copy(
            wt_hbm.at[:, pl.ds(tc * 128, 128)], blk_v.at[buf],
            sems[buf]).wait()
        blk = blk_v.at[buf]
        row0 = lax.iota(jnp.int32, 16)
        row1 = row0 + 16
        for q in range(32):
          for t in range(8):
            rows = row0 if t % 2 == 0 else row1
            col = jnp.full((16,), 4 * q + t // 2, jnp.int32)
            vals = plsc.load_gather(blk, [rows, col])
            chunk_v.at[q][pl.ds(t * 16, 16)] = vals
        pltpu.sync_copy(chunk_v, out_hbm.at[pl.ds(tc * 32, 32)])

    fire(0, 0)

    @pl.loop(0, _BPW, step=2)
    def _(i0):
      for b in range(2):
        i = i0 + b

        @pl.when(i + 1 < _BPW)
        def _():
          fire(i + 1, (b + 1) % 2)

        drain_transpose_store(i, b)

    @pl.when(wid == 0)
    def _():
      pltpu.sync_copy(tail_hbm, chunk_v.at[pl.ds(0, 16)])
      pltpu.sync_copy(
          chunk_v.at[pl.ds(0, 16)],
          out_hbm.at[pl.ds(_NBLK * 32, 16)])

  return k1


def _make_k2():
  mesh = plsc.VectorSubcoreMesh(core_axis_name="c", subcore_axis_name="s")

  @functools.partial(
      pl.kernel,
      out_type=jax.ShapeDtypeStruct((HIST, 4, _TCB, 8, 128), jnp.float32),
      mesh=mesh,
      compiler_params=pltpu.CompilerParams(
          use_tc_tiling_on_sc=False, needs_layout_passes=False),
      scratch_types=[
          pltpu.VMEM((_UPW * _U,), jnp.int32),
          pltpu.VMEM((2, _U, OUT_DIM), jnp.float32),
          pltpu.VMEM((32, 128), jnp.float32),
          pltpu.SemaphoreType.DMA,
          pltpu.SemaphoreType.DMA,
      ],
  )
  def k2(idx_hbm, table_hbm, out_hbm, idx_v, rows_v, chunk_v, sem0, sem1):
    wid = lax.axis_index("s") * _NC + lax.axis_index("c")
    ubase = wid * _UPW
    pltpu.sync_copy(idx_hbm.at[pl.ds(wid * _UPW * _U, _UPW * _U)], idx_v)

    sems = (sem0, sem1)

    def fire(u_local, buf):
      pltpu.async_copy(
          table_hbm.at[idx_v.at[pl.ds(u_local * _U, _U)]],
          rows_v.at[buf], sems[buf])

    def drain_transpose_store(u_local, buf):
      pltpu.make_async_copy(
          table_hbm.at[idx_v.at[pl.ds(u_local * _U, _U)]],
          rows_v.at[buf], sems[buf]).wait()
      rows = rows_v.at[buf]
      base = lax.iota(jnp.int32, 16)
      for c in range(OUT_DIM):
        col = jnp.full((16,), c, jnp.int32)
        for t in range(8):
          vals = plsc.load_gather(rows, [base + t * 16, col])
          chunk_v.at[c][pl.ds(t * 16, 16)] = vals

      u = ubase + u_local
      h = u // _TCB
      tc = lax.rem(u, _TCB)
      for tr in range(4):
        pltpu.sync_copy(
            chunk_v.at[pl.ds(tr * 8, 8)], out_hbm.at[h, tr, tc])

    fire(0, 0)

    @pl.loop(0, _UPW, step=2)
    def _(u0):
      for b in range(2):
        u_local = u0 + b

        @pl.when(u_local + 1 < _UPW)
        def _():
          fire(u_local + 1, (b + 1) % 2)

        drain_transpose_store(u_local, b)

  return k2


_transpose = _make_k1()
_gather = _make_k2()


@jax.jit
def kernel(net, weights):
  idx = net.T.reshape(_B).astype(jnp.int32)
  tail = weights[_NBLK * 128:].reshape(16, 128)
  table = _transpose(weights.T, tail)
  out5d = _gather(idx, table.reshape(IN_DIM, OUT_DIM))
  return out5d.transpose(2, 4, 0, 1, 3).reshape(BATCH, HIST, OUT_DIM)


# speedup vs baseline: 1.3672x; 1.3672x over previous
import functools

import jax
import jax.numpy as jnp
from jax import lax
from jax.experimental import pallas as pl
from jax.experimental.pallas import tpu as pltpu, tpu_sc as plsc

IN_DIM = 1_000_000
OUT_DIM = 32
BATCH = 4096
HIST = 50

_NC = 2
_NS = 16
_NW = _NC * _NS

_B = BATCH * HIST
_U = 128
_UPW = _B // (_NW * _U)
_TCB = BATCH // 128

_NBLK = IN_DIM // 128
_BPW = -(-_NBLK // _NW)
_LAST_COLS = IN_DIM - _NBLK * 128
_GROWS = IN_DIM * OUT_DIM // 128


def _make_k1():
  mesh = plsc.VectorSubcoreMesh(core_axis_name="c", subcore_axis_name="s")

  @functools.partial(
      pl.kernel,
      out_type=jax.ShapeDtypeStruct((_GROWS, 128), jnp.float32),
      mesh=mesh,
      compiler_params=pltpu.CompilerParams(needs_layout_passes=False),
      scratch_types=[
          pltpu.VMEM((2, 32, 128), jnp.float32),
          pltpu.VMEM((32, 128), jnp.float32),
          pltpu.SemaphoreType.DMA,
          pltpu.SemaphoreType.DMA,
      ],
  )
  def k1(wt_hbm, tail_hbm, out_hbm, blk_v, chunk_v, sem0, sem1):
    wid = lax.axis_index("s") * _NC + lax.axis_index("c")
    sems = (sem0, sem1)

    def fire(i, buf):
      tc = wid + i * _NW

      @pl.when(tc < _NBLK)
      def _():
        pltpu.async_copy(
            wt_hbm.at[:, pl.ds(tc * 128, 128)], blk_v.at[buf], sems[buf])

    def drain_transpose_store(i, buf):
      tc = wid + i * _NW

      @pl.when(tc < _NBLK)
      def _():
        pltpu.make_async_copy(
            wt_hbm.at[:, pl.ds(tc * 128, 128)], blk_v.at[buf],
            sems[buf]).wait()
        blk = blk_v.at[buf]
        row0 = lax.iota(jnp.int32, 16)
        row1 = row0 + 16
        for q in range(32):
          vals = [
              plsc.load_gather(
                  blk,
                  [row0 if t % 2 == 0 else row1,
                   jnp.full((16,), 4 * q + t // 2, jnp.int32)])
              for t in range(8)
          ]
          for t in range(8):
            chunk_v.at[q][pl.ds(t * 16, 16)] = vals[t]
        pltpu.sync_copy(chunk_v, out_hbm.at[pl.ds(tc * 32, 32)])

    fire(0, 0)

    @pl.loop(0, _BPW, step=2)
    def _(i0):
      for b in range(2):
        i = i0 + b

        @pl.when(i + 1 < _BPW)
        def _():
          fire(i + 1, (b + 1) % 2)

        drain_transpose_store(i, b)

    @pl.when(wid == 0)
    def _():
      pltpu.sync_copy(tail_hbm, chunk_v.at[pl.ds(0, 16)])
      pltpu.sync_copy(
          chunk_v.at[pl.ds(0, 16)],
          out_hbm.at[pl.ds(_NBLK * 32, 16)])

  return k1


def _make_k2():
  mesh = plsc.VectorSubcoreMesh(core_axis_name="c", subcore_axis_name="s")

  @functools.partial(
      pl.kernel,
      out_type=jax.ShapeDtypeStruct((HIST, 4, _TCB, 8, 128), jnp.float32),
      mesh=mesh,
      compiler_params=pltpu.CompilerParams(
          use_tc_tiling_on_sc=False, needs_layout_passes=False),
      scratch_types=[
          pltpu.VMEM((_UPW * _U,), jnp.int32),
          pltpu.VMEM((2, _U, OUT_DIM), jnp.float32),
          pltpu.VMEM((32, 128), jnp.float32),
          pltpu.SemaphoreType.DMA,
          pltpu.SemaphoreType.DMA,
      ],
  )
  def k2(idx_hbm, table_hbm, out_hbm, idx_v, rows_v, chunk_v, sem0, sem1):
    wid = lax.axis_index("s") * _NC + lax.axis_index("c")
    ubase = wid * _UPW
    pltpu.sync_copy(idx_hbm.at[pl.ds(wid * _UPW * _U, _UPW * _U)], idx_v)

    sems = (sem0, sem1)

    def fire(u_local, buf):
      pltpu.async_copy(
          table_hbm.at[idx_v.at[pl.ds(u_local * _U, _U)]],
          rows_v.at[buf], sems[buf])

    def drain_transpose_store(u_local, buf):
      pltpu.make_async_copy(
          table_hbm.at[idx_v.at[pl.ds(u_local * _U, _U)]],
          rows_v.at[buf], sems[buf]).wait()
      rows = rows_v.at[buf]
      base = lax.iota(jnp.int32, 16)
      for c in range(OUT_DIM):
        col = jnp.full((16,), c, jnp.int32)
        vals = [
            plsc.load_gather(rows, [base + t * 16, col])
            for t in range(8)
        ]
        for t in range(8):
          chunk_v.at[c][pl.ds(t * 16, 16)] = vals[t]

      u = ubase + u_local
      h = u // _TCB
      tc = lax.rem(u, _TCB)
      for tr in range(4):
        pltpu.sync_copy(
            chunk_v.at[pl.ds(tr * 8, 8)], out_hbm.at[h, tr, tc])

    fire(0, 0)

    @pl.loop(0, _UPW, step=2)
    def _(u0):
      for b in range(2):
        u_local = u0 + b

        @pl.when(u_local + 1 < _UPW)
        def _():
          fire(u_local + 1, (b + 1) % 2)

        drain_transpose_store(u_local, b)

  return k2


_transpose = _make_k1()
_gather = _make_k2()


@jax.jit
def kernel(net, weights):
  idx = net.T.reshape(_B).astype(jnp.int32)
  tail = weights[_NBLK * 128:].reshape(16, 128)
  table = _transpose(weights.T, tail)
  out5d = _gather(idx, table.reshape(IN_DIM, OUT_DIM))
  return out5d.transpose(2, 4, 0, 1, 3).reshape(BATCH, HIST, OUT_DIM)


# speedup vs baseline: 1.5248x; 1.1153x over previous
import functools

import jax
import jax.numpy as jnp
from jax import lax
from jax.experimental import pallas as pl
from jax.experimental.pallas import tpu as pltpu, tpu_sc as plsc

IN_DIM = 1_000_000
OUT_DIM = 32
BATCH = 4096
HIST = 50

_NC = 2
_NS = 16
_NW = _NC * _NS

_B = BATCH * HIST
_U = 128
_UPW = _B // (_NW * _U)
_TCB = BATCH // 128

_NBLK = IN_DIM // 128
_BPW = -(-_NBLK // _NW)
_LAST_COLS = IN_DIM - _NBLK * 128
_GROWS = IN_DIM * OUT_DIM // 128


def _make_k1():
  mesh = plsc.VectorSubcoreMesh(core_axis_name="c", subcore_axis_name="s")

  @functools.partial(
      pl.kernel,
      out_type=jax.ShapeDtypeStruct((_GROWS, 128), jnp.float32),
      mesh=mesh,
      compiler_params=pltpu.CompilerParams(needs_layout_passes=False),
      scratch_types=[
          pltpu.VMEM((2, 32, 128), jnp.float32),
          pltpu.VMEM((2, 32, 128), jnp.float32),
          pltpu.SemaphoreType.DMA,
          pltpu.SemaphoreType.DMA,
          pltpu.SemaphoreType.DMA,
          pltpu.SemaphoreType.DMA,
      ],
  )
  def k1(wt_hbm, tail_hbm, out_hbm, blk_v, chunk_v, semi0, semi1, semo0,
         semo1):
    wid = lax.axis_index("s") * _NC + lax.axis_index("c")
    semis = (semi0, semi1)
    semos = (semo0, semo1)

    def out_desc(i, buf):
      tc = wid + i * _NW
      return pltpu.make_async_copy(
          chunk_v.at[buf], out_hbm.at[pl.ds(tc * 32, 32)], semos[buf])

    def fire(i, buf):
      tc = wid + i * _NW

      @pl.when(tc < _NBLK)
      def _():
        pltpu.async_copy(
            wt_hbm.at[:, pl.ds(tc * 128, 128)], blk_v.at[buf], semis[buf])

    def drain_transpose_store(i, buf):
      tc = wid + i * _NW

      @pl.when(tc < _NBLK)
      def _():
        @pl.when(i >= 2)
        def _():
          out_desc(i - 2, buf).wait()

        pltpu.make_async_copy(
            wt_hbm.at[:, pl.ds(tc * 128, 128)], blk_v.at[buf],
            semis[buf]).wait()
        blk = blk_v.at[buf]
        row0 = lax.iota(jnp.int32, 16)
        row1 = row0 + 16

        def gathers(q):
          return [
              plsc.load_gather(
                  blk,
                  [row0 if t % 2 == 0 else row1,
                   jnp.full((16,), 4 * q + t // 2, jnp.int32)])
              for t in range(8)
          ]

        prev = gathers(0)
        for q in range(1, 32):
          cur = gathers(q)
          for t in range(8):
            chunk_v.at[buf].at[q - 1][pl.ds(t * 16, 16)] = prev[t]
          prev = cur
        for t in range(8):
          chunk_v.at[buf].at[31][pl.ds(t * 16, 16)] = prev[t]
        out_desc(i, buf).start()

    fire(0, 0)

    @pl.loop(0, _BPW, step=2)
    def _(i0):
      for b in range(2):
        i = i0 + b

        @pl.when(i + 1 < _BPW)
        def _():
          fire(i + 1, (b + 1) % 2)

        drain_transpose_store(i, b)

    for i in (_BPW - 2, _BPW - 1):
      tc = wid + i * _NW

      @pl.when(tc < _NBLK)
      def _():
        out_desc(i, i % 2).wait()

    @pl.when(wid == 0)
    def _():
      pltpu.sync_copy(tail_hbm, chunk_v.at[0].at[pl.ds(0, 16)])
      pltpu.sync_copy(
          chunk_v.at[0].at[pl.ds(0, 16)],
          out_hbm.at[pl.ds(_NBLK * 32, 16)])

  return k1


def _make_k2():
  mesh = plsc.VectorSubcoreMesh(core_axis_name="c", subcore_axis_name="s")

  @functools.partial(
      pl.kernel,
      out_type=jax.ShapeDtypeStruct((HIST, 4, _TCB, 8, 128), jnp.float32),
      mesh=mesh,
      compiler_params=pltpu.CompilerParams(
          use_tc_tiling_on_sc=False, needs_layout_passes=False),
      scratch_types=[
          pltpu.VMEM((_UPW * _U,), jnp.int32),
          pltpu.VMEM((2, _U, OUT_DIM), jnp.float32),
          pltpu.VMEM((2, 32, 128), jnp.float32),
          pltpu.SemaphoreType.DMA,
          pltpu.SemaphoreType.DMA,
          pltpu.SemaphoreType.DMA,
          pltpu.SemaphoreType.DMA,
      ],
  )
  def k2(idx_hbm, table_hbm, out_hbm, idx_v, rows_v, chunk_v, semi0, semi1,
         semo0, semo1):
    wid = lax.axis_index("s") * _NC + lax.axis_index("c")
    ubase = wid * _UPW
    pltpu.sync_copy(idx_hbm.at[pl.ds(wid * _UPW * _U, _UPW * _U)], idx_v)

    semis = (semi0, semi1)
    semos = (semo0, semo1)

    def out_descs(u_local, buf):
      u = ubase + u_local
      h = u // _TCB
      tc = lax.rem(u, _TCB)
      return [
          pltpu.make_async_copy(
              chunk_v.at[buf].at[pl.ds(tr * 8, 8)], out_hbm.at[h, tr, tc],
              semos[buf])
          for tr in range(4)
      ]

    def fire(u_local, buf):
      pltpu.async_copy(
          table_hbm.at[idx_v.at[pl.ds(u_local * _U, _U)]],
          rows_v.at[buf], semis[buf])

    def drain_transpose_store(u_local, buf):
      @pl.when(u_local >= 2)
      def _():
        for d in out_descs(u_local - 2, buf):
          d.wait()

      pltpu.make_async_copy(
          table_hbm.at[idx_v.at[pl.ds(u_local * _U, _U)]],
          rows_v.at[buf], semis[buf]).wait()
      rows = rows_v.at[buf]
      base = lax.iota(jnp.int32, 16)

      def gathers(c):
        col = jnp.full((16,), c, jnp.int32)
        return [
            plsc.load_gather(rows, [base + t * 16, col])
            for t in range(8)
        ]

      prev = gathers(0)
      for c in range(1, OUT_DIM):
        cur = gathers(c)
        for t in range(8):
          chunk_v.at[buf].at[c - 1][pl.ds(t * 16, 16)] = prev[t]
        prev = cur
      for t in range(8):
        chunk_v.at[buf].at[OUT_DIM - 1][pl.ds(t * 16, 16)] = prev[t]

      for d in out_descs(u_local, buf):
        d.start()

    fire(0, 0)

    @pl.loop(0, _UPW, step=2)
    def _(u0):
      for b in range(2):
        u_local = u0 + b

        @pl.when(u_local + 1 < _UPW)
        def _():
          fire(u_local + 1, (b + 1) % 2)

        drain_transpose_store(u_local, b)

    for u_local in (_UPW - 2, _UPW - 1):
      for d in out_descs(u_local, u_local % 2):
        d.wait()

  return k2


_transpose = _make_k1()
_gather = _make_k2()


@jax.jit
def kernel(net, weights):
  idx = net.T.reshape(_B).astype(jnp.int32)
  tail = weights[_NBLK * 128:].reshape(16, 128)
  table = _transpose(weights.T, tail)
  out5d = _gather(idx, table.reshape(IN_DIM, OUT_DIM))
  return out5d.transpose(2, 4, 0, 1, 3).reshape(BATCH, HIST, OUT_DIM)


# speedup vs baseline: 1.5938x; 1.0452x over previous
import functools

import jax
import jax.numpy as jnp
from jax import lax
from jax.experimental import pallas as pl
from jax.experimental.pallas import tpu as pltpu, tpu_sc as plsc

IN_DIM = 1_000_000
OUT_DIM = 32
BATCH = 4096
HIST = 50

_NC = 2
_NS = 16
_NW = _NC * _NS

_B = BATCH * HIST
_U = 128
_UPW = _B // (_NW * _U)
_TCB = BATCH // 128

_NBLK = IN_DIM // 128
_BPW = -(-_NBLK // _NW)
_LAST_COLS = IN_DIM - _NBLK * 128
_GROWS = IN_DIM * OUT_DIM // 128


def _make_k1():
  mesh = plsc.VectorSubcoreMesh(core_axis_name="c", subcore_axis_name="s")

  @functools.partial(
      pl.kernel,
      out_type=jax.ShapeDtypeStruct((_GROWS, 128), jnp.float32),
      mesh=mesh,
      compiler_params=pltpu.CompilerParams(needs_layout_passes=False),
      scratch_types=[
          pltpu.VMEM((2, 32, 128), jnp.float32),
          pltpu.VMEM((2, 32, 128), jnp.float32),
          pltpu.SemaphoreType.DMA,
          pltpu.SemaphoreType.DMA,
          pltpu.SemaphoreType.DMA,
          pltpu.SemaphoreType.DMA,
      ],
  )
  def k1(wt_hbm, tail_hbm, out_hbm, blk_v, chunk_v, semi0, semi1, semo0,
         semo1):
    wid = lax.axis_index("s") * _NC + lax.axis_index("c")
    semis = (semi0, semi1)
    semos = (semo0, semo1)

    def out_desc(i, buf):
      tc = wid + i * _NW
      return pltpu.make_async_copy(
          chunk_v.at[buf], out_hbm.at[pl.ds(tc * 32, 32)], semos[buf])

    def fire(i, buf):
      tc = wid + i * _NW

      @pl.when(tc < _NBLK)
      def _():
        pltpu.async_copy(
            wt_hbm.at[:, pl.ds(tc * 128, 128)], blk_v.at[buf], semis[buf])

    def drain_transpose_store(i, buf):
      tc = wid + i * _NW

      @pl.when(tc < _NBLK)
      def _():
        @pl.when(i >= 2)
        def _():
          out_desc(i - 2, buf).wait()

        pltpu.make_async_copy(
            wt_hbm.at[:, pl.ds(tc * 128, 128)], blk_v.at[buf],
            semis[buf]).wait()
        blk = blk_v.at[buf]
        chunk = chunk_v.at[buf]
        row0 = lax.iota(jnp.int32, 16)
        row1 = row0 + 16

        def gather(q, t):
          return plsc.load_gather(
              blk,
              [row0 if t % 2 == 0 else row1,
               jnp.full((16,), 4 * q + t // 2, jnp.int32)])

        prev = [gather(0, t) for t in range(8)]
        for q in range(1, 32):
          cur = []
          for t in range(8):
            cur.append(gather(q, t))
            chunk.at[q - 1][pl.ds(t * 16, 16)] = prev[t]
          prev = cur
        for t in range(8):
          chunk.at[31][pl.ds(t * 16, 16)] = prev[t]
        out_desc(i, buf).start()

    fire(0, 0)

    @pl.loop(0, _BPW, step=2)
    def _(i0):
      for b in range(2):
        i = i0 + b

        @pl.when(i + 1 < _BPW)
        def _():
          fire(i + 1, (b + 1) % 2)

        drain_transpose_store(i, b)

    for i in (_BPW - 2, _BPW - 1):
      tc = wid + i * _NW

      @pl.when(tc < _NBLK)
      def _():
        out_desc(i, i % 2).wait()

    @pl.when(wid == 0)
    def _():
      pltpu.sync_copy(tail_hbm, chunk_v.at[0].at[pl.ds(0, 16)])
      pltpu.sync_copy(
          chunk_v.at[0].at[pl.ds(0, 16)],
          out_hbm.at[pl.ds(_NBLK * 32, 16)])

  return k1


def _make_k2():
  mesh = plsc.VectorSubcoreMesh(core_axis_name="c", subcore_axis_name="s")

  @functools.partial(
      pl.kernel,
      out_type=jax.ShapeDtypeStruct((HIST, 4, _TCB, 8, 128), jnp.float32),
      mesh=mesh,
      compiler_params=pltpu.CompilerParams(
          use_tc_tiling_on_sc=False, needs_layout_passes=False),
      scratch_types=[
          pltpu.VMEM((_UPW * _U,), jnp.int32),
          pltpu.VMEM((2, _U, OUT_DIM), jnp.float32),
          pltpu.VMEM((2, 32, 128), jnp.float32),
          pltpu.SemaphoreType.DMA,
          pltpu.SemaphoreType.DMA,
          pltpu.SemaphoreType.DMA,
          pltpu.SemaphoreType.DMA,
      ],
  )
  def k2(idx_hbm, table_hbm, out_hbm, idx_v, rows_v, chunk_v, semi0, semi1,
         semo0, semo1):
    wid = lax.axis_index("s") * _NC + lax.axis_index("c")
    ubase = wid * _UPW
    pltpu.sync_copy(idx_hbm.at[pl.ds(wid * _UPW * _U, _UPW * _U)], idx_v)

    semis = (semi0, semi1)
    semos = (semo0, semo1)

    def out_descs(u_local, buf):
      u = ubase + u_local
      h = u // _TCB
      tc = lax.rem(u, _TCB)
      return [
          pltpu.make_async_copy(
              chunk_v.at[buf].at[pl.ds(tr * 8, 8)], out_hbm.at[h, tr, tc],
              semos[buf])
          for tr in range(4)
      ]

    def fire(u_local, buf):
      pltpu.async_copy(
          table_hbm.at[idx_v.at[pl.ds(u_local * _U, _U)]],
          rows_v.at[buf], semis[buf])

    def drain_transpose_store(u_local, buf):
      @pl.when(u_local >= 2)
      def _():
        for d in out_descs(u_local - 2, buf):
          d.wait()

      pltpu.make_async_copy(
          table_hbm.at[idx_v.at[pl.ds(u_local * _U, _U)]],
          rows_v.at[buf], semis[buf]).wait()
      rows = rows_v.at[buf]
      base = lax.iota(jnp.int32, 16)

      def gathers(c):
        col = jnp.full((16,), c, jnp.int32)
        return [
            plsc.load_gather(rows, [base + t * 16, col])
            for t in range(8)
        ]

      prev = gathers(0)
      for c in range(1, OUT_DIM):
        cur = gathers(c)
        for t in range(8):
          chunk_v.at[buf].at[c - 1][pl.ds(t * 16, 16)] = prev[t]
        prev = cur
      for t in range(8):
        chunk_v.at[buf].at[OUT_DIM - 1][pl.ds(t * 16, 16)] = prev[t]

      for d in out_descs(u_local, buf):
        d.start()

    fire(0, 0)

    @pl.loop(0, _UPW, step=2)
    def _(u0):
      for b in range(2):
        u_local = u0 + b

        @pl.when(u_local + 1 < _UPW)
        def _():
          fire(u_local + 1, (b + 1) % 2)

        drain_transpose_store(u_local, b)

    for u_local in (_UPW - 2, _UPW - 1):
      for d in out_descs(u_local, u_local % 2):
        d.wait()

  return k2


_transpose = _make_k1()
_gather = _make_k2()


@jax.jit
def kernel(net, weights):
  idx = net.T.reshape(_B).astype(jnp.int32)
  tail = weights[_NBLK * 128:].reshape(16, 128)
  table = _transpose(weights.T, tail)
  out5d = _gather(idx, table.reshape(IN_DIM, OUT_DIM))
  return out5d.transpose(2, 4, 0, 1, 3).reshape(BATCH, HIST, OUT_DIM)
